# baseline (speedup 1.0000x reference)
import functools

import jax
import jax.numpy as jnp
from jax import lax
from jax.experimental import pallas as pl
from jax.experimental.pallas import tpu as pltpu
from jax.experimental.pallas import tpu_sc as plsc

VOCAB = 100000
EMB = 32
CTX = 20
BATCH = 1024

_NC = 2
_NS = 16
_NW = _NC * _NS
_CHUNK = 128

_N_IDX = BATCH * CTX
_ROWS = _N_IDX // _CHUNK
_ROWS_PER_W = _ROWS // _NW


def _sc_gather(emb_table, idx3d):
    mesh = plsc.VectorSubcoreMesh(core_axis_name="c", subcore_axis_name="s")

    @functools.partial(
        pl.kernel,
        mesh=mesh,
        out_type=jax.ShapeDtypeStruct((_NW, _ROWS_PER_W, _CHUNK, EMB), jnp.float32),
        scratch_types=[
            pltpu.VMEM((_ROWS_PER_W, _CHUNK), jnp.int32),
            pltpu.VMEM((_ROWS_PER_W, _CHUNK, EMB), jnp.float32),
            pltpu.SemaphoreType.DMA,
        ],
        compiler_params=pltpu.CompilerParams(use_tc_tiling_on_sc=False),
    )
    def k(table_hbm, idx_hbm, out_hbm, idx_v, rows_v, sem):
        wid = lax.axis_index("s") * _NC + lax.axis_index("c")
        pltpu.sync_copy(idx_hbm.at[wid], idx_v)
        copies = [
            pltpu.async_copy(table_hbm.at[idx_v.at[j]], rows_v.at[j], sem)
            for j in range(_ROWS_PER_W)
        ]
        for c in copies:
            c.wait()
        pltpu.sync_copy(rows_v, out_hbm.at[wid])

    return k(emb_table, idx3d)


_VB = 2048


def _proj_body(xf_ref, w_ref, b_ref, out_ref):
    acc = lax.dot_general(
        xf_ref[...].astype(jnp.bfloat16),
        w_ref[...].astype(jnp.bfloat16),
        dimension_numbers=(((1,), (1,)), ((), ())),
        preferred_element_type=jnp.float32,
    )
    out_ref[...] = acc + b_ref[...]


def _tc_project(xf, W, b2d):
    n_blocks = pl.cdiv(VOCAB, _VB)
    return pl.pallas_call(
        _proj_body,
        grid=(n_blocks,),
        in_specs=[
            pl.BlockSpec((BATCH, CTX * EMB), lambda j: (0, 0)),
            pl.BlockSpec((_VB, CTX * EMB), lambda j: (j, 0)),
            pl.BlockSpec((1, _VB), lambda j: (0, j)),
        ],
        out_specs=pl.BlockSpec((BATCH, _VB), lambda j: (0, j)),
        out_shape=jax.ShapeDtypeStruct((BATCH, VOCAB), jnp.float32),
        compiler_params=pltpu.CompilerParams(
            dimension_semantics=("parallel",),
        ),
    )(xf, W, b2d)


def kernel(x, emb_table, W, b):
    idx3d = x.reshape(_NW, _ROWS_PER_W, _CHUNK).astype(jnp.int32)
    rows = _sc_gather(emb_table, idx3d)
    xf = rows.reshape(BATCH, CTX * EMB)
    return _tc_project(xf, W, b.reshape(1, VOCAB))
